# Initial kernel scaffold; baseline (speedup 1.0000x reference)
import jax
import jax.numpy as jnp
from jax.experimental import pallas as pl


def kernel(inputs, label, epoch, em):
    raise NotImplementedError("write your pallas kernel here")


# speedup vs baseline: 3.7718x; 3.7718x over previous
import functools

import jax
import jax.numpy as jnp
from jax.experimental import pallas as pl
from jax.experimental.pallas import tpu as pltpu

_BATCH = 128
_FEATURES = 2048
_CLASSES = 16522
_BETA = 0.05
_KNN = 6

_TILE_C = 1024
_NTILES = (_CLASSES + _TILE_C - 1) // _TILE_C
_NEG = -1e30


def _body(x_ref, em_ref, lab_ref, out_ref, m_s, s_s, vlab_s, topk_s):
    i = pl.program_id(0)

    @pl.when(i == 0)
    def _init():
        m_s[...] = jnp.full((_BATCH, 1), _NEG, jnp.float32)
        s_s[...] = jnp.zeros((_BATCH, 1), jnp.float32)
        vlab_s[...] = jnp.zeros((_BATCH, 1), jnp.float32)
        topk_s[...] = jnp.full((_BATCH, 8), _NEG, jnp.float32)

    scores = jax.lax.dot_general(
        x_ref[...], em_ref[...],
        (((1,), (1,)), ((), ())),
        preferred_element_type=jnp.float32,
    ) * (1.0 / _BETA)

    col = i * _TILE_C + jax.lax.broadcasted_iota(jnp.int32, (_BATCH, _TILE_C), 1)
    scores = jnp.where(col < _CLASSES, scores, _NEG)

    tmax = jnp.max(scores, axis=1, keepdims=True)
    m_prev = m_s[...]
    m_new = jnp.maximum(m_prev, tmax)
    s_s[...] = s_s[...] * jnp.exp(m_prev - m_new) + jnp.sum(
        jnp.exp(scores - m_new), axis=1, keepdims=True)
    m_s[...] = m_new

    vlab_s[...] += jnp.sum(
        jnp.where(col == lab_ref[...], scores, 0.0), axis=1, keepdims=True)

    cur = scores
    tile_top = []
    mj = tmax
    for j in range(_KNN):
        if j > 0:
            mj = jnp.max(cur, axis=1, keepdims=True)
        tile_top.append(mj)
        if j < _KNN - 1:
            cur = jnp.where(cur == mj, _NEG, cur)
    merged = jnp.concatenate([topk_s[...]] + tile_top, axis=1)
    new_top = []
    for _ in range(_KNN):
        mj = jnp.max(merged, axis=1, keepdims=True)
        new_top.append(mj)
        merged = jnp.where(merged == mj, _NEG, merged)
    pad = jnp.full((_BATCH, 8 - _KNN), _NEG, jnp.float32)
    topk_s[...] = jnp.concatenate(new_top + [pad], axis=1)

    @pl.when(i == _NTILES - 1)
    def _finish():
        lse = m_s[...] + jnp.log(s_s[...])
        top = topk_s[...]
        top_sum = jnp.sum(top[:, 0:_KNN], axis=1, keepdims=True)
        vlab = vlab_s[...]
        kth = top[:, _KNN - 1:_KNN]
        in_top = vlab >= kth
        s_dot = 2.0 * top_sum + jnp.where(in_top, vlab, 3.0 * vlab)
        m_tot = jnp.where(in_top, 13.0, 15.0)
        loss_rows = lse * m_tot - s_dot
        out_ref[0, 0] = jnp.sum(loss_rows) / _BATCH


@jax.jit
def _run(inputs, label, em):
    lab2d = label.reshape(_BATCH, 1).astype(jnp.int32)
    out = pl.pallas_call(
        _body,
        grid=(_NTILES,),
        in_specs=[
            pl.BlockSpec((_BATCH, _FEATURES), lambda i: (0, 0)),
            pl.BlockSpec((_TILE_C, _FEATURES), lambda i: (i, 0)),
            pl.BlockSpec((_BATCH, 1), lambda i: (0, 0)),
        ],
        out_specs=pl.BlockSpec(memory_space=pltpu.SMEM),
        out_shape=jax.ShapeDtypeStruct((1, 1), jnp.float32),
        scratch_shapes=[
            pltpu.VMEM((_BATCH, 1), jnp.float32),
            pltpu.VMEM((_BATCH, 1), jnp.float32),
            pltpu.VMEM((_BATCH, 1), jnp.float32),
            pltpu.VMEM((_BATCH, 8), jnp.float32),
        ],
    )(inputs, em, lab2d)
    return out[0, 0]


def kernel(inputs, label, epoch, em):
    del epoch
    return _run(inputs, label, em)


# speedup vs baseline: 4.0876x; 1.0837x over previous
import functools

import jax
import jax.numpy as jnp
from jax.experimental import pallas as pl
from jax.experimental.pallas import tpu as pltpu

_BATCH = 128
_FEATURES = 2048
_CLASSES = 16522
_BETA = 0.05
_KNN = 6

_TILE_C = 2048
_NTILES = (_CLASSES + _TILE_C - 1) // _TILE_C
_NEG = -1e30


def _body(x_ref, em_ref, lab_ref, out_ref, m_s, s_s, vlab_s, topk_s):
    i = pl.program_id(0)

    @pl.when(i == 0)
    def _init():
        m_s[...] = jnp.full((_BATCH, 1), _NEG, jnp.float32)
        s_s[...] = jnp.zeros((_BATCH, 1), jnp.float32)
        vlab_s[...] = jnp.zeros((_BATCH, 1), jnp.float32)
        topk_s[...] = jnp.full((_BATCH, 8), _NEG, jnp.float32)

    scores = jax.lax.dot_general(
        x_ref[...], em_ref[...],
        (((1,), (1,)), ((), ())),
        preferred_element_type=jnp.float32,
    ) * (1.0 / _BETA)

    col = i * _TILE_C + jax.lax.broadcasted_iota(jnp.int32, (_BATCH, _TILE_C), 1)
    scores = jnp.where(col < _CLASSES, scores, _NEG)

    tmax = jnp.max(scores, axis=1, keepdims=True)
    m_prev = m_s[...]
    m_new = jnp.maximum(m_prev, tmax)
    s_s[...] = s_s[...] * jnp.exp(m_prev - m_new) + jnp.sum(
        jnp.exp(scores - m_new), axis=1, keepdims=True)
    m_s[...] = m_new

    vlab_s[...] += jnp.sum(
        jnp.where(col == lab_ref[...], scores, 0.0), axis=1, keepdims=True)

    cur = scores
    tile_top = []
    mj = tmax
    for j in range(_KNN):
        if j > 0:
            mj = jnp.max(cur, axis=1, keepdims=True)
        tile_top.append(mj)
        if j < _KNN - 1:
            cur = jnp.where(cur == mj, _NEG, cur)
    merged = jnp.concatenate([topk_s[...]] + tile_top, axis=1)
    new_top = []
    for _ in range(_KNN):
        mj = jnp.max(merged, axis=1, keepdims=True)
        new_top.append(mj)
        merged = jnp.where(merged == mj, _NEG, merged)
    pad = jnp.full((_BATCH, 8 - _KNN), _NEG, jnp.float32)
    topk_s[...] = jnp.concatenate(new_top + [pad], axis=1)

    @pl.when(i == _NTILES - 1)
    def _finish():
        lse = m_s[...] + jnp.log(s_s[...])
        top = topk_s[...]
        top_sum = jnp.sum(top[:, 0:_KNN], axis=1, keepdims=True)
        vlab = vlab_s[...]
        kth = top[:, _KNN - 1:_KNN]
        in_top = vlab >= kth
        s_dot = 2.0 * top_sum + jnp.where(in_top, vlab, 3.0 * vlab)
        m_tot = jnp.where(in_top, 13.0, 15.0)
        loss_rows = lse * m_tot - s_dot
        out_ref[0, 0] = jnp.sum(loss_rows) / _BATCH


@jax.jit
def _run(inputs, label, em):
    lab2d = label.reshape(_BATCH, 1).astype(jnp.int32)
    out = pl.pallas_call(
        _body,
        grid=(_NTILES,),
        in_specs=[
            pl.BlockSpec((_BATCH, _FEATURES), lambda i: (0, 0)),
            pl.BlockSpec((_TILE_C, _FEATURES), lambda i: (i, 0)),
            pl.BlockSpec((_BATCH, 1), lambda i: (0, 0)),
        ],
        out_specs=pl.BlockSpec(memory_space=pltpu.SMEM),
        out_shape=jax.ShapeDtypeStruct((1, 1), jnp.float32),
        scratch_shapes=[
            pltpu.VMEM((_BATCH, 1), jnp.float32),
            pltpu.VMEM((_BATCH, 1), jnp.float32),
            pltpu.VMEM((_BATCH, 1), jnp.float32),
            pltpu.VMEM((_BATCH, 8), jnp.float32),
        ],
    )(inputs, em, lab2d)
    return out[0, 0]


def kernel(inputs, label, epoch, em):
    del epoch
    return _run(inputs, label, em)
